# baseline (speedup 1.0000x reference)
import functools

import jax
import jax.numpy as jnp
from jax import lax
from jax.experimental import pallas as pl
from jax.experimental.pallas import tpu as pltpu
from jax.experimental.pallas import tpu_sc as plsc

N = 10000
E = 320000
D = 128
G = 64

NC = 2
NS = 16
NW = NC * NS

CHUNK = 128
NCHUNKS = 80
EPW = NCHUNKS * CHUNK
E_PAD = EPW * NW
NBUF = 2
NGROUPS = NCHUNKS // NBUF
N_ACC = 10240
ROWS_PER_TILE_ACC = N_ACC // NS
ZCHUNK = 32

_sc_mesh = plsc.VectorSubcoreMesh(
    core_axis_name="c", subcore_axis_name="s", num_cores=NC, num_subcores=NS)


@functools.partial(
    pl.kernel,
    out_type=jax.ShapeDtypeStruct((NC, N_ACC), jnp.float32),
    mesh=_sc_mesh,
    scratch_types=[
        pltpu.VMEM_SHARED((N_ACC,), jnp.float32),
        pltpu.VMEM((CHUNK,), jnp.int32),
        pltpu.VMEM((CHUNK,), jnp.float32),
        pltpu.VMEM((ROWS_PER_TILE_ACC,), jnp.float32),
    ],
)
def _deg_kernel(idx_hbm, out_hbm, acc, cidx, ones_v, zbuf):
    ci = lax.axis_index("c")
    si = lax.axis_index("s")
    wid = si * NC + ci
    for i in range(CHUNK // 16):
        ones_v[pl.ds(i * 16, 16)] = jnp.ones((16,), jnp.float32)
    for i in range(ROWS_PER_TILE_ACC // 16):
        zbuf[pl.ds(i * 16, 16)] = jnp.zeros((16,), jnp.float32)
    pltpu.sync_copy(zbuf, acc.at[pl.ds(si * ROWS_PER_TILE_ACC, ROWS_PER_TILE_ACC)])
    plsc.subcore_barrier()

    def body(j, carry):
        pltpu.sync_copy(idx_hbm.at[wid, j, 1], cidx)
        pltpu.sync_copy(ones_v, acc.at[cidx], add=True)
        return carry

    lax.fori_loop(0, NCHUNKS, body, 0)
    plsc.subcore_barrier()
    pltpu.sync_copy(acc.at[pl.ds(si * ROWS_PER_TILE_ACC, ROWS_PER_TILE_ACC)],
                    out_hbm.at[ci, pl.ds(si * ROWS_PER_TILE_ACC, ROWS_PER_TILE_ACC)])


@functools.partial(
    pl.kernel,
    out_type=jax.ShapeDtypeStruct((NC, N_ACC, D), jnp.float32),
    mesh=_sc_mesh,
    scratch_types=[
        pltpu.VMEM_SHARED((N_ACC, D), jnp.float32),
        pltpu.VMEM((NBUF, 2, CHUNK), jnp.int32),
        pltpu.VMEM((NBUF, CHUNK, D), jnp.float32),
        pltpu.VMEM((ZCHUNK, D), jnp.float32),
    ] + [pltpu.SemaphoreType.DMA] * NBUF,
)
def _agg_kernel(y_hbm, idx_hbm, out_hbm, acc, idx, rows, zbuf, *sems):
    ci = lax.axis_index("c")
    si = lax.axis_index("s")
    wid = si * NC + ci
    for i in range(ZCHUNK * D // 16):
        zbuf[i // (D // 16), pl.ds((i % (D // 16)) * 16, 16)] = (
            jnp.zeros((16,), jnp.float32))
    for j in range(ROWS_PER_TILE_ACC // ZCHUNK):
        pltpu.sync_copy(
            zbuf, acc.at[pl.ds(si * ROWS_PER_TILE_ACC + j * ZCHUNK, ZCHUNK)])
    plsc.subcore_barrier()

    def load_and_gather(chunk, b):
        pltpu.sync_copy(idx_hbm.at[wid, chunk], idx.at[b])
        pltpu.async_copy(y_hbm.at[idx.at[b, 0]], rows.at[b], sems[b])

    for b in range(NBUF):
        load_and_gather(b, b)

    def body(g, carry):
        for b in range(NBUF):
            chunk = g * NBUF + b
            pltpu.make_async_copy(y_hbm.at[idx.at[b, 0]], rows.at[b],
                                  sems[b]).wait()
            pltpu.sync_copy(rows.at[b], acc.at[idx.at[b, 1]], add=True)

            @pl.when(g < NGROUPS - 1)
            def _():
                load_and_gather(chunk + NBUF, b)
        return carry

    lax.fori_loop(0, NGROUPS, body, 0)
    plsc.subcore_barrier()
    pltpu.sync_copy(acc.at[pl.ds(si * ROWS_PER_TILE_ACC, ROWS_PER_TILE_ACC)],
                    out_hbm.at[ci, pl.ds(si * ROWS_PER_TILE_ACC, ROWS_PER_TILE_ACC)])


def _rsqrt(u):
    r = lax.rsqrt(u)
    return r * (1.5 - 0.5 * u * r * r)


def _tc1_body(deg_ref, x_ref, w_ref, dinv_ref, y_ref):
    deg = deg_ref[0, :N] + deg_ref[1, :N] + 1.0
    dinv = _rsqrt(deg)[:, None]
    dinv_ref[...] = dinv
    y_ref[...] = dinv * jnp.dot(x_ref[...], w_ref[...],
                                preferred_element_type=jnp.float32, precision=lax.Precision.DEFAULT)


_tc1 = pl.pallas_call(
    _tc1_body,
    out_shape=(jax.ShapeDtypeStruct((N, 1), jnp.float32),
               jax.ShapeDtypeStruct((N, D), jnp.float32)),
)


def _bn_relu(z, g_ref, be_ref):
    m = jnp.mean(z, axis=0, keepdims=True)
    v = jnp.mean((z - m) ** 2, axis=0, keepdims=True)
    return jnp.maximum((z - m) * _rsqrt(v + 1e-5) * g_ref[...] + be_ref[...], 0.0)


def _tc_layer_body(p_ref, y_ref, dinv_ref, b_ref, g_ref, be_ref, w_ref, ynext_ref):
    agg = p_ref[0, :N] + p_ref[1, :N]
    z = dinv_ref[...] * (agg + y_ref[...]) + b_ref[...]
    h = _bn_relu(z, g_ref, be_ref)
    ynext_ref[...] = dinv_ref[...] * jnp.dot(h, w_ref[...],
                                             preferred_element_type=jnp.float32, precision=lax.Precision.DEFAULT)


_tc_layer = pl.pallas_call(
    _tc_layer_body,
    out_shape=jax.ShapeDtypeStruct((N, D), jnp.float32),
)


def _tc_final_body(p_ref, y_ref, dinv_ref, b_ref, g_ref, be_ref, batch_ref,
                   m1w_ref, m1b_ref, m2w_ref, m2b_ref, out_ref):
    agg = p_ref[0, :N] + p_ref[1, :N]
    z = dinv_ref[...] * (agg + y_ref[...]) + b_ref[...]
    h = _bn_relu(z, g_ref, be_ref)
    seg = lax.broadcasted_iota(jnp.int32, (1, G), 1)
    onehot = jnp.where(batch_ref[...] == seg, 1.0, 0.0)
    pooled = lax.dot_general(onehot, h, (((0,), (0,)), ((), ())),
                             preferred_element_type=jnp.float32, precision=lax.Precision.DEFAULT)
    q = jnp.maximum(jnp.dot(pooled, m1w_ref[...],
                            preferred_element_type=jnp.float32, precision=lax.Precision.DEFAULT) + m1b_ref[...], 0.0)
    out_ref[...] = jnp.dot(q, m2w_ref[...],
                           preferred_element_type=jnp.float32, precision=lax.Precision.DEFAULT) + m2b_ref[...]


_tc_final = pl.pallas_call(
    _tc_final_body,
    out_shape=jax.ShapeDtypeStruct((G, 1), jnp.float32),
)


def kernel(x, edge_index, batch, W1, b1, g1, be1, W2, b2, g2, be2,
           W3, b3, g3, be3, M1w, M1b, M2w, M2b):
    row = edge_index[0].astype(jnp.int32)
    col = edge_index[1].astype(jnp.int32)
    npad = E_PAD - E
    rp = jnp.concatenate([row, jnp.zeros((npad,), jnp.int32)])
    cp = jnp.concatenate(
        [col, N + jnp.arange(npad, dtype=jnp.int32) % (N_ACC - N)])
    idx_pack = jnp.stack(
        [rp.reshape(NW, NCHUNKS, CHUNK), cp.reshape(NW, NCHUNKS, CHUNK)],
        axis=2)

    deg_parts = _deg_kernel(idx_pack)
    dinv, y1 = _tc1(deg_parts, x, W1)

    b1r, g1r, be1r = b1[None, :], g1[None, :], be1[None, :]
    b2r, g2r, be2r = b2[None, :], g2[None, :], be2[None, :]
    b3r, g3r, be3r = b3[None, :], g3[None, :], be3[None, :]

    p = _agg_kernel(y1, idx_pack)
    y2 = _tc_layer(p, y1, dinv, b1r, g1r, be1r, W2)
    p = _agg_kernel(y2, idx_pack)
    y3 = _tc_layer(p, y2, dinv, b2r, g2r, be2r, W3)
    p = _agg_kernel(y3, idx_pack)
    out = _tc_final(p, y3, dinv, b3r, g3r, be3r, batch[:, None].astype(jnp.int32),
                    M1w, M1b[None, :], M2w, M2b[None, :])
    return out


# speedup vs baseline: 1.8243x; 1.8243x over previous
import functools

import jax
import jax.numpy as jnp
from jax import lax
from jax.experimental import pallas as pl
from jax.experimental.pallas import tpu as pltpu
from jax.experimental.pallas import tpu_sc as plsc

N = 10000
E = 320000
D = 128
G = 64

NC = 2
NS = 16
NW = NC * NS
DH = D // NC

CHUNK = 128
E_PAD = 327680
NBUF = 4

NCH_DEG = E_PAD // (NW * CHUNK)
NCH_AGG = E_PAD // (NS * CHUNK)
NG_AGG = NCH_AGG // NBUF

N_ACC = 10240
RPT = N_ACC // NS
ZCHUNK = 32

_sc_mesh = plsc.VectorSubcoreMesh(
    core_axis_name="c", subcore_axis_name="s", num_cores=NC, num_subcores=NS)


@functools.partial(
    pl.kernel,
    out_type=jax.ShapeDtypeStruct((NC, N_ACC), jnp.float32),
    mesh=_sc_mesh,
    scratch_types=[
        pltpu.VMEM_SHARED((N_ACC,), jnp.float32),
        pltpu.VMEM((CHUNK,), jnp.int32),
        pltpu.VMEM((CHUNK,), jnp.float32),
        pltpu.VMEM((RPT,), jnp.float32),
    ],
)
def _deg_kernel(idx_hbm, out_hbm, acc, cidx, ones_v, zbuf):
    ci = lax.axis_index("c")
    si = lax.axis_index("s")
    wid = si * NC + ci
    for i in range(CHUNK // 16):
        ones_v[pl.ds(i * 16, 16)] = jnp.ones((16,), jnp.float32)
    for i in range(RPT // 16):
        zbuf[pl.ds(i * 16, 16)] = jnp.zeros((16,), jnp.float32)
    pltpu.sync_copy(zbuf, acc.at[pl.ds(si * RPT, RPT)])
    plsc.subcore_barrier()

    def body(j, carry):
        pltpu.sync_copy(idx_hbm.at[wid, j, 1], cidx)
        pltpu.sync_copy(ones_v, acc.at[cidx], add=True)
        return carry

    lax.fori_loop(0, NCH_DEG, body, 0)
    plsc.subcore_barrier()
    pltpu.sync_copy(acc.at[pl.ds(si * RPT, RPT)],
                    out_hbm.at[ci, pl.ds(si * RPT, RPT)])


@functools.partial(
    pl.kernel,
    out_type=jax.ShapeDtypeStruct((NC, N_ACC, DH), jnp.float32),
    mesh=_sc_mesh,
    scratch_types=[
        pltpu.VMEM_SHARED((N_ACC, DH), jnp.float32),
        pltpu.VMEM_SHARED((N_ACC, DH), jnp.float32),
        pltpu.VMEM((NBUF, 2, CHUNK), jnp.int32),
        pltpu.VMEM((NBUF, CHUNK, DH), jnp.float32),
        pltpu.VMEM((ZCHUNK, DH), jnp.float32),
    ] + [pltpu.SemaphoreType.DMA] * NBUF,
    compiler_params=pltpu.CompilerParams(use_tc_tiling_on_sc=False),
)
def _agg_kernel(ysplit_hbm, idx_hbm, out_hbm, ysp, acc, idx, rows, zbuf, *sems):
    ci = lax.axis_index("c")
    si = lax.axis_index("s")
    for i in range(ZCHUNK * DH // 16):
        zbuf[i // (DH // 16), pl.ds((i % (DH // 16)) * 16, 16)] = (
            jnp.zeros((16,), jnp.float32))
    for j in range(RPT // ZCHUNK):
        pltpu.sync_copy(zbuf, acc.at[pl.ds(si * RPT + j * ZCHUNK, ZCHUNK)])
    pltpu.sync_copy(ysplit_hbm.at[ci, pl.ds(si * RPT, RPT)],
                    ysp.at[pl.ds(si * RPT, RPT)])
    plsc.subcore_barrier()

    def load_and_gather(chunk, b):
        pltpu.sync_copy(idx_hbm.at[si, chunk], idx.at[b])
        pltpu.async_copy(ysp.at[idx.at[b, 0]], rows.at[b], sems[b])

    for b in range(NBUF):
        load_and_gather(b, b)

    def body(g, carry):
        for b in range(NBUF):
            chunk = g * NBUF + b
            pltpu.make_async_copy(ysp.at[idx.at[b, 0]], rows.at[b],
                                  sems[b]).wait()
            pltpu.sync_copy(rows.at[b], acc.at[idx.at[b, 1]], add=True)

            @pl.when(g < NG_AGG - 1)
            def _():
                load_and_gather(chunk + NBUF, b)
        return carry

    lax.fori_loop(0, NG_AGG, body, 0)
    plsc.subcore_barrier()
    pltpu.sync_copy(acc.at[pl.ds(si * RPT, RPT)],
                    out_hbm.at[ci, pl.ds(si * RPT, RPT)])


def _rsqrt(u):
    r = lax.rsqrt(u)
    return r * (1.5 - 0.5 * u * r * r)


def _merge(ref):
    return jnp.concatenate([ref[0, :N], ref[1, :N]], axis=1)


def _split_write(ref, val):
    ref[0, :N] = val[:, :DH]
    ref[1, :N] = val[:, DH:]


def _tc1_body(deg_ref, x_ref, w_ref, dinv_ref, y_ref):
    deg = deg_ref[0, :N] + deg_ref[1, :N] + 1.0
    dinv = _rsqrt(deg)[:, None]
    dinv_ref[...] = dinv
    _split_write(y_ref, dinv * jnp.dot(x_ref[...], w_ref[...],
                                       preferred_element_type=jnp.float32))


_tc1 = pl.pallas_call(
    _tc1_body,
    out_shape=(jax.ShapeDtypeStruct((N, 1), jnp.float32),
               jax.ShapeDtypeStruct((NC, N_ACC, DH), jnp.float32)),
)


def _bn_relu(z, g_ref, be_ref):
    m = jnp.mean(z, axis=0, keepdims=True)
    v = jnp.mean((z - m) ** 2, axis=0, keepdims=True)
    return jnp.maximum((z - m) * _rsqrt(v + 1e-5) * g_ref[...] + be_ref[...], 0.0)


def _tc_layer_body(p_ref, y_ref, dinv_ref, b_ref, g_ref, be_ref, w_ref, ynext_ref):
    z = dinv_ref[...] * (_merge(p_ref) + _merge(y_ref)) + b_ref[...]
    h = _bn_relu(z, g_ref, be_ref)
    _split_write(ynext_ref, dinv_ref[...] * jnp.dot(
        h, w_ref[...], preferred_element_type=jnp.float32))


_tc_layer = pl.pallas_call(
    _tc_layer_body,
    out_shape=jax.ShapeDtypeStruct((NC, N_ACC, DH), jnp.float32),
)


def _tc_final_body(p_ref, y_ref, dinv_ref, b_ref, g_ref, be_ref, batch_ref,
                   m1w_ref, m1b_ref, m2w_ref, m2b_ref, out_ref):
    z = dinv_ref[...] * (_merge(p_ref) + _merge(y_ref)) + b_ref[...]
    h = _bn_relu(z, g_ref, be_ref)
    seg = lax.broadcasted_iota(jnp.int32, (1, G), 1)
    onehot = jnp.where(batch_ref[...] == seg, 1.0, 0.0)
    pooled = lax.dot_general(onehot, h, (((0,), (0,)), ((), ())),
                             preferred_element_type=jnp.float32)
    q = jnp.maximum(jnp.dot(pooled, m1w_ref[...],
                            preferred_element_type=jnp.float32) + m1b_ref[...],
                    0.0)
    out_ref[...] = jnp.dot(q, m2w_ref[...],
                           preferred_element_type=jnp.float32) + m2b_ref[...]


_tc_final = pl.pallas_call(
    _tc_final_body,
    out_shape=jax.ShapeDtypeStruct((G, 1), jnp.float32),
)


def kernel(x, edge_index, batch, W1, b1, g1, be1, W2, b2, g2, be2,
           W3, b3, g3, be3, M1w, M1b, M2w, M2b):
    row = edge_index[0].astype(jnp.int32)
    col = edge_index[1].astype(jnp.int32)
    npad = E_PAD - E
    rp = jnp.concatenate([row, jnp.zeros((npad,), jnp.int32)])
    cp = jnp.concatenate(
        [col, N + jnp.arange(npad, dtype=jnp.int32) % (N_ACC - N)])
    idx_deg = jnp.stack(
        [rp.reshape(NW, NCH_DEG, CHUNK), cp.reshape(NW, NCH_DEG, CHUNK)],
        axis=2)
    idx_agg = jnp.stack(
        [rp.reshape(NS, NCH_AGG, CHUNK), cp.reshape(NS, NCH_AGG, CHUNK)],
        axis=2)

    deg_parts = _deg_kernel(idx_deg)
    dinv, y1 = _tc1(deg_parts, x, W1)

    b1r, g1r, be1r = b1[None, :], g1[None, :], be1[None, :]
    b2r, g2r, be2r = b2[None, :], g2[None, :], be2[None, :]
    b3r, g3r, be3r = b3[None, :], g3[None, :], be3[None, :]

    p = _agg_kernel(y1, idx_agg)
    y2 = _tc_layer(p, y1, dinv, b1r, g1r, be1r, W2)
    p = _agg_kernel(y2, idx_agg)
    y3 = _tc_layer(p, y2, dinv, b2r, g2r, be2r, W3)
    p = _agg_kernel(y3, idx_agg)
    out = _tc_final(p, y3, dinv, b3r, g3r, be3r, batch[:, None].astype(jnp.int32),
                    M1w, M1b[None, :], M2w, M2b[None, :])
    return out
